# baseline (speedup 1.0000x reference)
import jax
import jax.numpy as jnp
from jax import lax
from jax.experimental import pallas as pl
from jax.experimental.pallas import tpu as pltpu
from jax.experimental.pallas import tpu_sc as plsc

D_MODEL = 1024
SEQ = 2048
BATCH = 4
NUM_CORES = 2
NUM_SUBCORES = 16
NUM_WORKERS = NUM_CORES * NUM_SUBCORES
SEQ_PER_WORKER = SEQ // NUM_WORKERS
CHUNK = 8
NUM_CHUNKS = SEQ_PER_WORKER // CHUNK
LANES = 16
VECS_PER_ROW = D_MODEL // LANES


def _body(x_hbm, emb_hbm, pe_hbm, out_hbm,
          idx0, idx1, ebuf0, ebuf1, xbuf0, xbuf1,
          gsem0, gsem1, isem0, isem1, osem0, osem1):
    idx = [idx0, idx1]
    ebuf = [ebuf0, ebuf1]
    xbuf = [xbuf0, xbuf1]
    gsem = [gsem0, gsem1]
    isem = [isem0, isem1]
    osem = [osem0, osem1]

    wid = lax.axis_index("s") * NUM_CORES + lax.axis_index("c")
    base = wid * SEQ_PER_WORKER

    def start_chunk(c):
        slot = c % 2
        pltpu.sync_copy(pe_hbm.at[pl.ds(base + c * CHUNK, CHUNK)], idx[slot])
        g = pltpu.async_copy(emb_hbm.at[idx[slot]], ebuf[slot], gsem[slot])
        i = pltpu.async_copy(
            x_hbm.at[:, pl.ds(base + c * CHUNK, CHUNK)], xbuf[slot],
            isem[slot])
        return g, i

    g_desc = [None, None]
    in_desc = [None, None]
    out_desc = [None, None]

    g_desc[0], in_desc[0] = start_chunk(0)

    for c in range(NUM_CHUNKS):
        cur = c % 2
        if c + 1 < NUM_CHUNKS:
            nxt = (c + 1) % 2
            if out_desc[nxt] is not None:
                out_desc[nxt].wait()
                out_desc[nxt] = None
            g_desc[nxt], in_desc[nxt] = start_chunk(c + 1)
        in_desc[cur].wait()
        g_desc[cur].wait()
        eb = ebuf[cur]
        xb = xbuf[cur]

        def batch_add(b, _):
            def row_add(r, _):
                for o in range(VECS_PER_ROW):
                    plsc.addupdate(
                        xb.at[b, r, pl.ds(o * LANES, LANES)],
                        eb[r, pl.ds(o * LANES, LANES)],
                    )
                return 0

            lax.fori_loop(0, CHUNK, row_add, 0)
            return 0

        lax.fori_loop(0, BATCH, batch_add, 0)
        out_desc[cur] = pltpu.async_copy(
            xb, out_hbm.at[:, pl.ds(base + c * CHUNK, CHUNK)], osem[cur])

    for d in out_desc:
        if d is not None:
            d.wait()


def kernel(x, emb, pe):
    mesh = plsc.VectorSubcoreMesh(
        core_axis_name="c",
        subcore_axis_name="s",
        num_cores=NUM_CORES,
        num_subcores=NUM_SUBCORES,
    )
    run = pl.kernel(
        _body,
        out_type=jax.ShapeDtypeStruct((BATCH, SEQ, D_MODEL), jnp.float32),
        mesh=mesh,
        scratch_types=[
            pltpu.VMEM((CHUNK,), jnp.int32),
            pltpu.VMEM((CHUNK,), jnp.int32),
            pltpu.VMEM((CHUNK, D_MODEL), jnp.float32),
            pltpu.VMEM((CHUNK, D_MODEL), jnp.float32),
            pltpu.VMEM((BATCH, CHUNK, D_MODEL), jnp.float32),
            pltpu.VMEM((BATCH, CHUNK, D_MODEL), jnp.float32),
            pltpu.SemaphoreType.DMA,
            pltpu.SemaphoreType.DMA,
            pltpu.SemaphoreType.DMA,
            pltpu.SemaphoreType.DMA,
            pltpu.SemaphoreType.DMA,
            pltpu.SemaphoreType.DMA,
        ],
        name="learnt_pos_enc_sc",
    )
    return run(x, emb, pe)


# speedup vs baseline: 1.5345x; 1.5345x over previous
import jax
import jax.numpy as jnp
from jax import lax
from jax.experimental import pallas as pl
from jax.experimental.pallas import tpu as pltpu
from jax.experimental.pallas import tpu_sc as plsc

D_MODEL = 1024
SEQ = 2048
BATCH = 4
NUM_CORES = 2
NUM_SUBCORES = 16
NUM_WORKERS = NUM_CORES * NUM_SUBCORES
SEQ_PER_WORKER = SEQ // NUM_WORKERS
CHUNK = 16
NUM_CHUNKS = SEQ_PER_WORKER // CHUNK
NUM_ITEMS = NUM_CHUNKS * BATCH
NBUF = 3
LANES = 16
VECS_PER_ROW = D_MODEL // LANES


def _body(x_hbm, emb_hbm, pe_hbm, out_hbm,
          idx0, idx1, ebuf0, ebuf1, xbuf0, xbuf1, xbuf2,
          gsem0, gsem1, isem0, isem1, isem2, osem0, osem1, osem2):
    idx = [idx0, idx1]
    ebuf = [ebuf0, ebuf1]
    xbuf = [xbuf0, xbuf1, xbuf2]
    gsem = [gsem0, gsem1]
    isem = [isem0, isem1, isem2]
    osem = [osem0, osem1, osem2]

    wid = lax.axis_index("s") * NUM_CORES + lax.axis_index("c")
    base = wid * SEQ_PER_WORKER

    def start_gather(c):
        slot = c % 2
        pltpu.sync_copy(pe_hbm.at[pl.ds(base + c * CHUNK, CHUNK)], idx[slot])
        return pltpu.async_copy(emb_hbm.at[idx[slot]], ebuf[slot], gsem[slot])

    def start_in(k):
        c, b = k // BATCH, k % BATCH
        return pltpu.async_copy(
            x_hbm.at[b, pl.ds(base + c * CHUNK, CHUNK)], xbuf[k % NBUF],
            isem[k % NBUF])

    g_desc = [None, None]
    in_desc = [None] * NBUF
    out_desc = [None] * NBUF

    g_desc[0] = start_gather(0)
    in_desc[0] = start_in(0)
    in_desc[1] = start_in(1)

    for k in range(NUM_ITEMS):
        c, b = k // BATCH, k % BATCH
        cur = k % NBUF
        if k + 2 < NUM_ITEMS:
            nxt = (k + 2) % NBUF
            if out_desc[nxt] is not None:
                out_desc[nxt].wait()
                out_desc[nxt] = None
            if (k + 2) % BATCH == 0:
                g_desc[((k + 2) // BATCH) % 2] = start_gather((k + 2) // BATCH)
            in_desc[nxt] = start_in(k + 2)
        in_desc[cur].wait()
        if b == 0:
            g_desc[c % 2].wait()
        eb = ebuf[c % 2]
        xb = xbuf[cur]

        def row_add(r, _):
            for o in range(VECS_PER_ROW):
                plsc.addupdate(
                    xb.at[r, pl.ds(o * LANES, LANES)],
                    eb[r, pl.ds(o * LANES, LANES)],
                )
            return 0

        lax.fori_loop(0, CHUNK, row_add, 0)
        out_desc[cur] = pltpu.async_copy(
            xb, out_hbm.at[b, pl.ds(base + c * CHUNK, CHUNK)], osem[cur])

    for d in out_desc:
        if d is not None:
            d.wait()


def kernel(x, emb, pe):
    mesh = plsc.VectorSubcoreMesh(
        core_axis_name="c",
        subcore_axis_name="s",
        num_cores=NUM_CORES,
        num_subcores=NUM_SUBCORES,
    )
    run = pl.kernel(
        _body,
        out_type=jax.ShapeDtypeStruct((BATCH, SEQ, D_MODEL), jnp.float32),
        mesh=mesh,
        scratch_types=[
            pltpu.VMEM((CHUNK,), jnp.int32),
            pltpu.VMEM((CHUNK,), jnp.int32),
            pltpu.VMEM((CHUNK, D_MODEL), jnp.float32),
            pltpu.VMEM((CHUNK, D_MODEL), jnp.float32),
            pltpu.VMEM((CHUNK, D_MODEL), jnp.float32),
            pltpu.VMEM((CHUNK, D_MODEL), jnp.float32),
            pltpu.VMEM((CHUNK, D_MODEL), jnp.float32),
            pltpu.SemaphoreType.DMA,
            pltpu.SemaphoreType.DMA,
            pltpu.SemaphoreType.DMA,
            pltpu.SemaphoreType.DMA,
            pltpu.SemaphoreType.DMA,
            pltpu.SemaphoreType.DMA,
            pltpu.SemaphoreType.DMA,
            pltpu.SemaphoreType.DMA,
        ],
        name="learnt_pos_enc_sc",
    )
    return run(x, emb, pe)


# speedup vs baseline: 1.5405x; 1.0039x over previous
import jax
import jax.numpy as jnp
from jax import lax
from jax.experimental import pallas as pl
from jax.experimental.pallas import tpu as pltpu
from jax.experimental.pallas import tpu_sc as plsc

D_MODEL = 1024
SEQ = 2048
BATCH = 4
NUM_CORES = 2
NUM_SUBCORES = 16
NUM_WORKERS = NUM_CORES * NUM_SUBCORES
SEQ_PER_WORKER = SEQ // NUM_WORKERS
CHUNK = 32
NUM_CHUNKS = SEQ_PER_WORKER // CHUNK
NUM_ITEMS = NUM_CHUNKS * BATCH
LANES = 16
VECS_PER_ROW = D_MODEL // LANES


def _body(x_hbm, emb_hbm, pe_hbm, out_hbm,
          idx0, idx1, ebuf, xbuf0, xbuf1,
          gsem, isem0, isem1, osem0, osem1):
    idx = [idx0, idx1]
    xbuf = [xbuf0, xbuf1]
    isem = [isem0, isem1]
    osem = [osem0, osem1]

    wid = lax.axis_index("s") * NUM_CORES + lax.axis_index("c")
    base = wid * SEQ_PER_WORKER

    def start_gather(c):
        pltpu.sync_copy(pe_hbm.at[pl.ds(base + c * CHUNK, CHUNK)], idx[c % 2])
        return pltpu.async_copy(emb_hbm.at[idx[c % 2]], ebuf, gsem)

    def start_in(k):
        c, b = k // BATCH, k % BATCH
        return pltpu.async_copy(
            x_hbm.at[b, pl.ds(base + c * CHUNK, CHUNK)], xbuf[k % 2],
            isem[k % 2])

    g_desc = start_gather(0)
    in_desc = [None, None]
    out_desc = [None, None]
    in_desc[0] = start_in(0)

    for k in range(NUM_ITEMS):
        c, b = k // BATCH, k % BATCH
        cur = k % 2
        if k + 1 < NUM_ITEMS:
            nxt = (k + 1) % 2
            if out_desc[nxt] is not None:
                out_desc[nxt].wait()
                out_desc[nxt] = None
            in_desc[nxt] = start_in(k + 1)
        in_desc[cur].wait()
        if b == 0:
            g_desc.wait()
        xb = xbuf[cur]

        def row_add(r, _):
            for o in range(VECS_PER_ROW):
                plsc.addupdate(
                    xb.at[r, pl.ds(o * LANES, LANES)],
                    ebuf[r, pl.ds(o * LANES, LANES)],
                )
            return 0

        lax.fori_loop(0, CHUNK, row_add, 0)
        if b == BATCH - 1 and c + 1 < NUM_CHUNKS:
            g_desc = start_gather(c + 1)
        out_desc[cur] = pltpu.async_copy(
            xb, out_hbm.at[b, pl.ds(base + c * CHUNK, CHUNK)], osem[cur])

    for d in out_desc:
        if d is not None:
            d.wait()


def kernel(x, emb, pe):
    mesh = plsc.VectorSubcoreMesh(
        core_axis_name="c",
        subcore_axis_name="s",
        num_cores=NUM_CORES,
        num_subcores=NUM_SUBCORES,
    )
    run = pl.kernel(
        _body,
        out_type=jax.ShapeDtypeStruct((BATCH, SEQ, D_MODEL), jnp.float32),
        mesh=mesh,
        scratch_types=[
            pltpu.VMEM((CHUNK,), jnp.int32),
            pltpu.VMEM((CHUNK,), jnp.int32),
            pltpu.VMEM((CHUNK, D_MODEL), jnp.float32),
            pltpu.VMEM((CHUNK, D_MODEL), jnp.float32),
            pltpu.VMEM((CHUNK, D_MODEL), jnp.float32),
            pltpu.SemaphoreType.DMA,
            pltpu.SemaphoreType.DMA,
            pltpu.SemaphoreType.DMA,
            pltpu.SemaphoreType.DMA,
            pltpu.SemaphoreType.DMA,
        ],
        name="learnt_pos_enc_sc",
    )
    return run(x, emb, pe)
